# baseline (speedup 1.0000x reference)
import functools

import jax
import jax.numpy as jnp
from jax import lax
from jax.experimental import pallas as pl
from jax.experimental.pallas import tpu as pltpu
from jax.experimental.pallas import tpu_sc as plsc

_C = 128
_NUM_WORKERS = 32


def _sc_edge_segment_sum(mhat, src3, dst3, zrows, n_pad, n_chunks):
  D = mhat.shape[1]
  rows_per_sub = n_pad // 16

  nbuf = 4
  assert n_chunks % nbuf == 0

  def body(h_hbm, src_hbm, dst_hbm, z_hbm, out_hbm,
           idx_s, idx_d, rows0, rows1, rows2, rows3, acc,
           sem0, sem1, sem2, sem3):
    rows = [rows0, rows1, rows2, rows3]
    sems = [sem0, sem1, sem2, sem3]
    c = lax.axis_index("c")
    s = lax.axis_index("s")
    w = s * 2 + c
    pltpu.sync_copy(src_hbm.at[w], idx_s)
    pltpu.sync_copy(dst_hbm.at[w], idx_d)
    pltpu.sync_copy(z_hbm, acc.at[pl.ds(s * rows_per_sub, rows_per_sub)])
    plsc.subcore_barrier()

    def chunk_simple(j, _):
      pltpu.async_copy(h_hbm.at[idx_s.at[j]], rows[0], sems[0]).wait()
      pltpu.sync_copy(rows[0], acc.at[idx_d.at[j]], add=True)
      return 0

    lax.fori_loop(0, n_chunks, chunk_simple, 0, unroll=False)
    plsc.subcore_barrier()
    pltpu.sync_copy(acc.at[pl.ds(s * rows_per_sub, rows_per_sub)],
                    out_hbm.at[c, pl.ds(s * rows_per_sub, rows_per_sub)])

  mesh = plsc.VectorSubcoreMesh(core_axis_name="c", subcore_axis_name="s")
  f = pl.kernel(
      body,
      out_type=jax.ShapeDtypeStruct((2, n_pad, D), jnp.float32),
      mesh=mesh,
      scratch_types=[
          pltpu.VMEM((n_chunks, _C), jnp.int32),
          pltpu.VMEM((n_chunks, _C), jnp.int32),
          pltpu.VMEM((_C, D), jnp.float32),
          pltpu.VMEM((_C, D), jnp.float32),
          pltpu.VMEM((_C, D), jnp.float32),
          pltpu.VMEM((_C, D), jnp.float32),
          pltpu.VMEM_SHARED((n_pad, D), jnp.float32),
          pltpu.SemaphoreType.DMA,
          pltpu.SemaphoreType.DMA,
          pltpu.SemaphoreType.DMA,
          pltpu.SemaphoreType.DMA,
      ],
  )
  return f(mhat, src3, dst3, zrows)


def _tc_matmul(x, W):
  NP, D = x.shape
  BN = 1024

  def body(x_ref, w_ref, o_ref):
    o_ref[...] = jnp.dot(x_ref[...], w_ref[...],
                         preferred_element_type=jnp.float32)

  return pl.pallas_call(
      body,
      grid=(NP // BN,),
      in_specs=[
          pl.BlockSpec((BN, D), lambda i: (i, 0)),
          pl.BlockSpec((D, D), lambda i: (0, 0)),
      ],
      out_specs=pl.BlockSpec((BN, D), lambda i: (i, 0)),
      out_shape=jax.ShapeDtypeStruct((NP, D), jnp.float32),
  )(x, W)


def _tc_relu_matmul(p2, b_prev, W):
  _, NP, D = p2.shape
  BN = 1024

  def body(p_ref, b_ref, w_ref, o_ref):
    h = jnp.maximum(p_ref[0] + p_ref[1] + b_ref[...], 0.0)
    o_ref[...] = jnp.dot(h, w_ref[...], preferred_element_type=jnp.float32)

  return pl.pallas_call(
      body,
      grid=(NP // BN,),
      in_specs=[
          pl.BlockSpec((2, BN, D), lambda i: (0, i, 0)),
          pl.BlockSpec((1, D), lambda i: (0, 0)),
          pl.BlockSpec((D, D), lambda i: (0, 0)),
      ],
      out_specs=pl.BlockSpec((BN, D), lambda i: (i, 0)),
      out_shape=jax.ShapeDtypeStruct((NP, D), jnp.float32),
  )(p2, b_prev.reshape(1, D), W)


def _tc_pool(p2, b1, Wp, bp, gid3, G):
  _, NP, D = p2.shape
  BN = 1024

  def body(p_ref, b1_ref, wp_ref, bp_ref, g_ref, o_ref):
    i = pl.program_id(0)
    h2 = jnp.maximum(p_ref[0] + p_ref[1] + b1_ref[...], 0.0)
    gr = jnp.dot(h2, wp_ref[...], preferred_element_type=jnp.float32) + bp_ref[...]
    gid = g_ref[...].reshape(BN)
    onehot_t = (lax.broadcasted_iota(jnp.int32, (G, BN), 0)
                == gid[None, :]).astype(jnp.float32)
    part = jnp.dot(onehot_t, gr, preferred_element_type=jnp.float32,
                   precision=lax.Precision.HIGHEST)

    @pl.when(i == 0)
    def _():
      o_ref[...] = jnp.zeros_like(o_ref)

    o_ref[...] += part

  return pl.pallas_call(
      body,
      grid=(NP // BN,),
      in_specs=[
          pl.BlockSpec((2, BN, D), lambda i: (0, i, 0)),
          pl.BlockSpec((1, D), lambda i: (0, 0)),
          pl.BlockSpec((D, D), lambda i: (0, 0)),
          pl.BlockSpec((1, D), lambda i: (0, 0)),
          pl.BlockSpec((1, 8, BN // 8), lambda i: (i, 0, 0)),
      ],
      out_specs=pl.BlockSpec((G, D), lambda i: (0, 0)),
      out_shape=jax.ShapeDtypeStruct((G, D), jnp.float32),
  )(p2, b1.reshape(1, D), Wp, bp.reshape(1, D), gid3)


def _tc_head(hGs, W_fan1, b_fan1, W_fan2, b_fan2,
             W_finit1, b_finit1, W_finit2, b_finit2):
  G, D = hGs.shape
  K2 = W_fan2.shape[1]

  def body(hg, wf1, bf1, wf2, bf2, wi1, bi1, wi2, bi2, p_out, hv_out):
    h = hg[...]
    fan_h = jax.nn.sigmoid(
        jnp.dot(h, wf1[...], preferred_element_type=jnp.float32) + bf1[...])
    logits = jnp.dot(fan_h, wf2[...], preferred_element_type=jnp.float32) + bf2[...]
    m = jnp.max(logits, axis=1, keepdims=True)
    e = jnp.exp(logits - m)
    p_out[...] = e / jnp.sum(e, axis=1, keepdims=True)
    fin_h = jax.nn.sigmoid(
        jnp.dot(h, wi1[...], preferred_element_type=jnp.float32) + bi1[...])
    hv_out[...] = jnp.dot(fin_h, wi2[...], preferred_element_type=jnp.float32) + bi2[...]

  return pl.pallas_call(
      body,
      out_shape=(jax.ShapeDtypeStruct((G, K2), jnp.float32),
                 jax.ShapeDtypeStruct((G, D), jnp.float32)),
  )(hGs, W_fan1, b_fan1.reshape(1, D), W_fan2, b_fan2.reshape(1, K2),
    W_finit1, b_finit1.reshape(1, D), W_finit2, b_finit2.reshape(1, D))


def kernel(x, edge_index, graph_ids, W_gcn, b_gcn, W_proj, b_proj,
           W_fan1, b_fan1, W_fan2, b_fan2, W_finit1, b_finit1,
           W_finit2, b_finit2):
  N, D = x.shape
  E = edge_index.shape[1]
  G = 64

  BN = 1024
  n_pad = ((N + BN - 1) // BN) * BN
  ew = -(-E // _NUM_WORKERS)
  n_chunks = -(-ew // _C)
  n_chunks = ((n_chunks + 3) // 4) * 4
  e_pad = _NUM_WORKERS * n_chunks * _C

  src = edge_index[0]
  dst = edge_index[1]
  pad = e_pad - E
  src_p = jnp.concatenate([src, jnp.zeros((pad,), jnp.int32)])
  dst_p = jnp.concatenate([dst, jnp.full((pad,), N, jnp.int32)])
  src3 = src_p.reshape(_NUM_WORKERS, n_chunks, _C)
  dst3 = dst_p.reshape(_NUM_WORKERS, n_chunks, _C)
  zrows = jnp.zeros((n_pad // 16, D), jnp.float32)

  gid_p = jnp.concatenate([graph_ids, jnp.full((n_pad - N,), G, jnp.int32)])
  gid3 = gid_p.reshape(n_pad // BN, 8, BN // 8)

  x_p = jnp.concatenate([x, jnp.zeros((n_pad - N, D), jnp.float32)])

  m0 = _tc_matmul(x_p, W_gcn[0])
  p0 = _sc_edge_segment_sum(m0, src3, dst3, zrows, n_pad, n_chunks)
  m1 = _tc_relu_matmul(p0, b_gcn[0], W_gcn[1])
  p1 = _sc_edge_segment_sum(m1, src3, dst3, zrows, n_pad, n_chunks)
  hGs = _tc_pool(p1, b_gcn[1], W_proj, b_proj, gid3, G)
  p, hvs = _tc_head(hGs, W_fan1, b_fan1, W_fan2, b_fan2,
                    W_finit1, b_finit1, W_finit2, b_finit2)
  return jnp.concatenate([p, hvs], axis=1)


# speedup vs baseline: 1.5023x; 1.5023x over previous
import functools

import jax
import jax.numpy as jnp
from jax import lax
from jax.experimental import pallas as pl
from jax.experimental.pallas import tpu as pltpu
from jax.experimental.pallas import tpu_sc as plsc

_C = 128
_NUM_WORKERS = 32


def _sc_edge_segment_sum(mhat, src3, dst3, zrows, n_pad, n_chunks):
  D = mhat.shape[1]
  rows_per_sub = n_pad // 16

  def body(h_hbm, src_hbm, dst_hbm, z_hbm, out_hbm,
           idx_s, idx_d, rows0, acc, sem0):
    c = lax.axis_index("c")
    s = lax.axis_index("s")
    w = s * 2 + c
    pltpu.sync_copy(src_hbm.at[w], idx_s)
    pltpu.sync_copy(dst_hbm.at[w], idx_d)
    pltpu.sync_copy(z_hbm, acc.at[pl.ds(s * rows_per_sub, rows_per_sub)])
    plsc.subcore_barrier()

    def chunk_simple(j, _):
      pltpu.async_copy(h_hbm.at[idx_s.at[j]], rows0, sem0).wait()
      pltpu.sync_copy(rows0, acc.at[idx_d.at[j]], add=True)
      return 0

    lax.fori_loop(0, n_chunks, chunk_simple, 0, unroll=False)
    plsc.subcore_barrier()
    pltpu.sync_copy(acc.at[pl.ds(s * rows_per_sub, rows_per_sub)],
                    out_hbm.at[c, pl.ds(s * rows_per_sub, rows_per_sub)])

  mesh = plsc.VectorSubcoreMesh(core_axis_name="c", subcore_axis_name="s")
  f = pl.kernel(
      body,
      out_type=jax.ShapeDtypeStruct((2, n_pad, D), jnp.float32),
      mesh=mesh,
      scratch_types=[
          pltpu.VMEM((n_chunks, _C), jnp.int32),
          pltpu.VMEM((n_chunks, _C), jnp.int32),
          pltpu.VMEM((_C, D), jnp.float32),
          pltpu.VMEM_SHARED((n_pad, D), jnp.float32),
          pltpu.SemaphoreType.DMA,
      ],
  )
  return f(mhat, src3, dst3, zrows)


def _tc_matmul(x, W):
  NP, D = x.shape
  BN = 1024

  def body(x_ref, w_ref, o_ref):
    o_ref[...] = jnp.dot(x_ref[...], w_ref[...],
                         preferred_element_type=jnp.float32)

  return pl.pallas_call(
      body,
      grid=(NP // BN,),
      in_specs=[
          pl.BlockSpec((BN, D), lambda i: (i, 0)),
          pl.BlockSpec((D, D), lambda i: (0, 0)),
      ],
      out_specs=pl.BlockSpec((BN, D), lambda i: (i, 0)),
      out_shape=jax.ShapeDtypeStruct((NP, D), jnp.float32),
  )(x, W)


def _tc_relu_matmul(p2, b_prev, W):
  _, NP, D = p2.shape
  BN = 1024

  def body(p_ref, b_ref, w_ref, o_ref):
    h = jnp.maximum(p_ref[0] + p_ref[1] + b_ref[...], 0.0)
    o_ref[...] = jnp.dot(h, w_ref[...], preferred_element_type=jnp.float32)

  return pl.pallas_call(
      body,
      grid=(NP // BN,),
      in_specs=[
          pl.BlockSpec((2, BN, D), lambda i: (0, i, 0)),
          pl.BlockSpec((1, D), lambda i: (0, 0)),
          pl.BlockSpec((D, D), lambda i: (0, 0)),
      ],
      out_specs=pl.BlockSpec((BN, D), lambda i: (i, 0)),
      out_shape=jax.ShapeDtypeStruct((NP, D), jnp.float32),
  )(p2, b_prev.reshape(1, D), W)


def _tc_pool(p2, b1, Wp, bp, gid3, G):
  _, NP, D = p2.shape
  BN = 1024

  def body(p_ref, b1_ref, wp_ref, bp_ref, g_ref, o_ref):
    i = pl.program_id(0)
    h2 = jnp.maximum(p_ref[0] + p_ref[1] + b1_ref[...], 0.0)
    gr = jnp.dot(h2, wp_ref[...], preferred_element_type=jnp.float32) + bp_ref[...]
    gid = g_ref[...].reshape(BN)
    onehot_t = (lax.broadcasted_iota(jnp.int32, (G, BN), 0)
                == gid[None, :]).astype(jnp.float32)
    part = jnp.dot(onehot_t, gr, preferred_element_type=jnp.float32,
                   precision=lax.Precision.HIGHEST)

    @pl.when(i == 0)
    def _():
      o_ref[...] = jnp.zeros_like(o_ref)

    o_ref[...] += part

  return pl.pallas_call(
      body,
      grid=(NP // BN,),
      in_specs=[
          pl.BlockSpec((2, BN, D), lambda i: (0, i, 0)),
          pl.BlockSpec((1, D), lambda i: (0, 0)),
          pl.BlockSpec((D, D), lambda i: (0, 0)),
          pl.BlockSpec((1, D), lambda i: (0, 0)),
          pl.BlockSpec((1, 8, BN // 8), lambda i: (i, 0, 0)),
      ],
      out_specs=pl.BlockSpec((G, D), lambda i: (0, 0)),
      out_shape=jax.ShapeDtypeStruct((G, D), jnp.float32),
  )(p2, b1.reshape(1, D), Wp, bp.reshape(1, D), gid3)


def _tc_head(hGs, W_fan1, b_fan1, W_fan2, b_fan2,
             W_finit1, b_finit1, W_finit2, b_finit2):
  G, D = hGs.shape
  K2 = W_fan2.shape[1]

  def body(hg, wf1, bf1, wf2, bf2, wi1, bi1, wi2, bi2, p_out, hv_out):
    h = hg[...]
    fan_h = jax.nn.sigmoid(
        jnp.dot(h, wf1[...], preferred_element_type=jnp.float32) + bf1[...])
    logits = jnp.dot(fan_h, wf2[...], preferred_element_type=jnp.float32) + bf2[...]
    m = jnp.max(logits, axis=1, keepdims=True)
    e = jnp.exp(logits - m)
    p_out[...] = e / jnp.sum(e, axis=1, keepdims=True)
    fin_h = jax.nn.sigmoid(
        jnp.dot(h, wi1[...], preferred_element_type=jnp.float32) + bi1[...])
    hv_out[...] = jnp.dot(fin_h, wi2[...], preferred_element_type=jnp.float32) + bi2[...]

  return pl.pallas_call(
      body,
      out_shape=(jax.ShapeDtypeStruct((G, K2), jnp.float32),
                 jax.ShapeDtypeStruct((G, D), jnp.float32)),
  )(hGs, W_fan1, b_fan1.reshape(1, D), W_fan2, b_fan2.reshape(1, K2),
    W_finit1, b_finit1.reshape(1, D), W_finit2, b_finit2.reshape(1, D))


def kernel(x, edge_index, graph_ids, W_gcn, b_gcn, W_proj, b_proj,
           W_fan1, b_fan1, W_fan2, b_fan2, W_finit1, b_finit1,
           W_finit2, b_finit2):
  N, D = x.shape
  E = edge_index.shape[1]
  G = 64

  BN = 1024
  n_pad = ((N + BN - 1) // BN) * BN
  ew = -(-E // _NUM_WORKERS)
  n_chunks = -(-ew // _C)
  e_pad = _NUM_WORKERS * n_chunks * _C

  src = edge_index[0]
  dst = edge_index[1]
  pad = e_pad - E
  src_p = jnp.concatenate([src, jnp.zeros((pad,), jnp.int32)])
  dst_p = jnp.concatenate([dst, jnp.full((pad,), N, jnp.int32)])
  src3 = src_p.reshape(_NUM_WORKERS, n_chunks, _C)
  dst3 = dst_p.reshape(_NUM_WORKERS, n_chunks, _C)
  zrows = jnp.zeros((n_pad // 16, D), jnp.float32)

  gid_p = jnp.concatenate([graph_ids, jnp.full((n_pad - N,), G, jnp.int32)])
  gid3 = gid_p.reshape(n_pad // BN, 8, BN // 8)

  x_p = jnp.concatenate([x, jnp.zeros((n_pad - N, D), jnp.float32)])

  m0 = _tc_matmul(x_p, W_gcn[0])
  p0 = _sc_edge_segment_sum(m0, src3, dst3, zrows, n_pad, n_chunks)
  m1 = _tc_relu_matmul(p0, b_gcn[0], W_gcn[1])
  p1 = _sc_edge_segment_sum(m1, src3, dst3, zrows, n_pad, n_chunks)
  hGs = _tc_pool(p1, b_gcn[1], W_proj, b_proj, gid3, G)
  p, hvs = _tc_head(hGs, W_fan1, b_fan1, W_fan2, b_fan2,
                    W_finit1, b_finit1, W_finit2, b_finit2)
  return jnp.concatenate([p, hvs], axis=1)


# speedup vs baseline: 1.5023x; 1.0000x over previous
import jax
import jax.numpy as jnp
from jax import lax
from jax.experimental import pallas as pl
from jax.experimental.pallas import tpu as pltpu
from jax.experimental.pallas import tpu_sc as plsc

_C = 128
_NUM_WORKERS = 32


def _sc_edge_segment_sum(mhat, src3, dst3, zrows, n_pad, n_chunks):
  D = mhat.shape[1]
  rows_per_sub = n_pad // 16

  def body(h_hbm, src_hbm, dst_hbm, z_hbm, out_hbm,
           idx_s, idx_d, rows0, acc, sem0):
    c = lax.axis_index("c")
    s = lax.axis_index("s")
    w = s * 2 + c
    pltpu.sync_copy(src_hbm.at[w], idx_s)
    pltpu.sync_copy(dst_hbm.at[w], idx_d)
    pltpu.sync_copy(z_hbm, acc.at[pl.ds(s * rows_per_sub, rows_per_sub)])
    plsc.subcore_barrier()

    def chunk_body(j, _):
      pltpu.async_copy(h_hbm.at[idx_s.at[j]], rows0, sem0).wait()
      pltpu.sync_copy(rows0, acc.at[idx_d.at[j]], add=True)
      return 0

    lax.fori_loop(0, n_chunks, chunk_body, 0, unroll=False)
    plsc.subcore_barrier()
    pltpu.sync_copy(acc.at[pl.ds(s * rows_per_sub, rows_per_sub)],
                    out_hbm.at[c, pl.ds(s * rows_per_sub, rows_per_sub)])

  mesh = plsc.VectorSubcoreMesh(core_axis_name="c", subcore_axis_name="s")
  f = pl.kernel(
      body,
      out_type=jax.ShapeDtypeStruct((2, n_pad, D), jnp.float32),
      mesh=mesh,
      scratch_types=[
          pltpu.VMEM((n_chunks, _C), jnp.int32),
          pltpu.VMEM((n_chunks, _C), jnp.int32),
          pltpu.VMEM((_C, D), jnp.float32),
          pltpu.VMEM_SHARED((n_pad, D), jnp.float32),
          pltpu.SemaphoreType.DMA,
      ],
  )
  return f(mhat, src3, dst3, zrows)


def _tc_matmul(x, W):
  NP, D = x.shape
  BN = 1024

  def body(x_ref, w_ref, o_ref):
    o_ref[...] = jnp.dot(x_ref[...], w_ref[...],
                         preferred_element_type=jnp.float32)

  return pl.pallas_call(
      body,
      grid=(NP // BN,),
      in_specs=[
          pl.BlockSpec((BN, D), lambda i: (i, 0)),
          pl.BlockSpec((D, D), lambda i: (0, 0)),
      ],
      out_specs=pl.BlockSpec((BN, D), lambda i: (i, 0)),
      out_shape=jax.ShapeDtypeStruct((NP, D), jnp.float32),
  )(x, W)


def _tc_relu_matmul(p2, b_prev, W):
  _, NP, D = p2.shape
  BN = 1024

  def body(p_ref, b_ref, w_ref, o_ref):
    h = jnp.maximum(p_ref[0] + p_ref[1] + b_ref[...], 0.0)
    o_ref[...] = jnp.dot(h, w_ref[...], preferred_element_type=jnp.float32)

  return pl.pallas_call(
      body,
      grid=(NP // BN,),
      in_specs=[
          pl.BlockSpec((2, BN, D), lambda i: (0, i, 0)),
          pl.BlockSpec((1, D), lambda i: (0, 0)),
          pl.BlockSpec((D, D), lambda i: (0, 0)),
      ],
      out_specs=pl.BlockSpec((BN, D), lambda i: (i, 0)),
      out_shape=jax.ShapeDtypeStruct((NP, D), jnp.float32),
  )(p2, b_prev.reshape(1, D), W)


def _tc_pool(p2, b1, Wp, bp, gid3, G):
  _, NP, D = p2.shape
  BN = 1024

  def body(p_ref, b1_ref, wp_ref, bp_ref, g_ref, o_ref):
    i = pl.program_id(0)
    h2 = jnp.maximum(p_ref[0] + p_ref[1] + b1_ref[...], 0.0)
    gr = jnp.dot(h2, wp_ref[...], preferred_element_type=jnp.float32) + bp_ref[...]
    gid = g_ref[...].reshape(BN)
    onehot_t = (lax.broadcasted_iota(jnp.int32, (G, BN), 0)
                == gid[None, :]).astype(jnp.float32)
    part = jnp.dot(onehot_t, gr, preferred_element_type=jnp.float32,
                   precision=lax.Precision.HIGHEST)

    @pl.when(i == 0)
    def _():
      o_ref[...] = jnp.zeros_like(o_ref)

    o_ref[...] += part

  return pl.pallas_call(
      body,
      grid=(NP // BN,),
      in_specs=[
          pl.BlockSpec((2, BN, D), lambda i: (0, i, 0)),
          pl.BlockSpec((1, D), lambda i: (0, 0)),
          pl.BlockSpec((D, D), lambda i: (0, 0)),
          pl.BlockSpec((1, D), lambda i: (0, 0)),
          pl.BlockSpec((1, 8, BN // 8), lambda i: (i, 0, 0)),
      ],
      out_specs=pl.BlockSpec((G, D), lambda i: (0, 0)),
      out_shape=jax.ShapeDtypeStruct((G, D), jnp.float32),
  )(p2, b1.reshape(1, D), Wp, bp.reshape(1, D), gid3)


def _tc_head(hGs, W_fan1, b_fan1, W_fan2, b_fan2,
             W_finit1, b_finit1, W_finit2, b_finit2):
  G, D = hGs.shape
  K2 = W_fan2.shape[1]

  def body(hg, wf1, bf1, wf2, bf2, wi1, bi1, wi2, bi2, p_out, hv_out):
    h = hg[...]
    fan_h = jax.nn.sigmoid(
        jnp.dot(h, wf1[...], preferred_element_type=jnp.float32) + bf1[...])
    logits = jnp.dot(fan_h, wf2[...], preferred_element_type=jnp.float32) + bf2[...]
    m = jnp.max(logits, axis=1, keepdims=True)
    e = jnp.exp(logits - m)
    p_out[...] = e / jnp.sum(e, axis=1, keepdims=True)
    fin_h = jax.nn.sigmoid(
        jnp.dot(h, wi1[...], preferred_element_type=jnp.float32) + bi1[...])
    hv_out[...] = jnp.dot(fin_h, wi2[...], preferred_element_type=jnp.float32) + bi2[...]

  return pl.pallas_call(
      body,
      out_shape=(jax.ShapeDtypeStruct((G, K2), jnp.float32),
                 jax.ShapeDtypeStruct((G, D), jnp.float32)),
  )(hGs, W_fan1, b_fan1.reshape(1, D), W_fan2, b_fan2.reshape(1, K2),
    W_finit1, b_finit1.reshape(1, D), W_finit2, b_finit2.reshape(1, D))


def kernel(x, edge_index, graph_ids, W_gcn, b_gcn, W_proj, b_proj,
           W_fan1, b_fan1, W_fan2, b_fan2, W_finit1, b_finit1,
           W_finit2, b_finit2):
  N, D = x.shape
  E = edge_index.shape[1]
  G = 64

  BN = 1024
  n_pad = ((N + BN - 1) // BN) * BN
  ew = -(-E // _NUM_WORKERS)
  n_chunks = -(-ew // _C)
  e_pad = _NUM_WORKERS * n_chunks * _C

  src = edge_index[0]
  dst = edge_index[1]
  pad = e_pad - E
  src_p = jnp.concatenate([src, jnp.zeros((pad,), jnp.int32)])
  dst_p = jnp.concatenate([dst, jnp.full((pad,), N, jnp.int32)])
  src3 = src_p.reshape(_NUM_WORKERS, n_chunks, _C)
  dst3 = dst_p.reshape(_NUM_WORKERS, n_chunks, _C)
  zrows = jnp.zeros((n_pad // 16, D), jnp.float32)

  gid_p = jnp.concatenate([graph_ids, jnp.full((n_pad - N,), G, jnp.int32)])
  gid3 = gid_p.reshape(n_pad // BN, 8, BN // 8)

  x_p = jnp.concatenate([x, jnp.zeros((n_pad - N, D), jnp.float32)])

  m0 = _tc_matmul(x_p, W_gcn[0])
  p0 = _sc_edge_segment_sum(m0, src3, dst3, zrows, n_pad, n_chunks)
  m1 = _tc_relu_matmul(p0, b_gcn[0], W_gcn[1])
  p1 = _sc_edge_segment_sum(m1, src3, dst3, zrows, n_pad, n_chunks)
  hGs = _tc_pool(p1, b_gcn[1], W_proj, b_proj, gid3, G)
  p, hvs = _tc_head(hGs, W_fan1, b_fan1, W_fan2, b_fan2,
                    W_finit1, b_finit1, W_finit2, b_finit2)
  return jnp.concatenate([p, hvs], axis=1)
